# baseline (speedup 1.0000x reference)
import jax
import jax.numpy as jnp
from jax.experimental import pallas as pl
from jax.experimental.pallas import tpu as pltpu

N = 100000
H = 64
D = 128
BLK = 6400
NBLK = (N + BLK - 1) // BLK


def _fused_kernel(adjt_ref, emb_ref, out_ref, stash_ref, lat_ref):
    i = pl.program_id(0)
    j = pl.program_id(1)

    @pl.when(jnp.logical_and(i == 0, j == 0))
    def _init():
        lat_ref[...] = jnp.zeros_like(lat_ref)

    @pl.when(jnp.logical_and(i == 0, j < NBLK - 1))
    def _accumulate():
        a = adjt_ref[...].astype(jnp.bfloat16)
        e = emb_ref[...].astype(jnp.bfloat16)
        stash_ref[j] = a
        lat_ref[...] += jnp.dot(a, e, preferred_element_type=jnp.float32)

    @pl.when(jnp.logical_and(i == 0, j == NBLK - 1))
    def _accumulate_tail():
        n_valid = N - (NBLK - 1) * BLK
        a = adjt_ref[...].astype(jnp.bfloat16)
        e = emb_ref[...].astype(jnp.bfloat16)
        acol = jax.lax.broadcasted_iota(jnp.int32, (H, BLK), 1)
        erow = jax.lax.broadcasted_iota(jnp.int32, (BLK, D), 0)
        a = jnp.where(acol < n_valid, a, jnp.bfloat16(0))
        e = jnp.where(erow < n_valid, e, jnp.bfloat16(0))
        stash_ref[j] = a
        lat_ref[...] += jnp.dot(a, e, preferred_element_type=jnp.float32)

    @pl.when(i == 1)
    def _emit():
        out_ref[...] = jax.lax.dot_general(
            stash_ref[j], lat_ref[...].astype(jnp.bfloat16),
            dimension_numbers=(((0,), (0,)), ((), ())),
            preferred_element_type=jnp.float32,
        )


def kernel(adj, embeds):
    adjt = jnp.swapaxes(adj, 0, 1)
    ret = pl.pallas_call(
        _fused_kernel,
        grid=(2, NBLK),
        in_specs=[
            pl.BlockSpec((H, BLK), lambda i, j: (0, (1 - i) * j + i * (NBLK - 1))),
            pl.BlockSpec((BLK, D), lambda i, j: ((1 - i) * j + i * (NBLK - 1), 0)),
        ],
        out_specs=pl.BlockSpec((BLK, D), lambda i, j: (i * j, 0)),
        out_shape=jax.ShapeDtypeStruct((N, D), jnp.float32),
        scratch_shapes=[
            pltpu.VMEM((NBLK, H, BLK), jnp.bfloat16),
            pltpu.VMEM((H, D), jnp.float32),
        ],
    )(adjt, embeds)
    return ret


# speedup vs baseline: 1.0341x; 1.0341x over previous
import jax
import jax.numpy as jnp
from jax.experimental import pallas as pl
from jax.experimental.pallas import tpu as pltpu

N = 100000
H = 64
D = 128
BLK = 16384
NBLK = (N + BLK - 1) // BLK


def _fused_kernel(adjt_ref, emb_ref, out_ref, stash_ref, lat_ref):
    i = pl.program_id(0)
    j = pl.program_id(1)

    @pl.when(jnp.logical_and(i == 0, j == 0))
    def _init():
        lat_ref[...] = jnp.zeros_like(lat_ref)

    @pl.when(jnp.logical_and(i == 0, j < NBLK - 1))
    def _accumulate():
        a = adjt_ref[...].astype(jnp.bfloat16)
        e = emb_ref[...].astype(jnp.bfloat16)
        stash_ref[j] = a
        lat_ref[...] += jnp.dot(a, e, preferred_element_type=jnp.float32)

    @pl.when(jnp.logical_and(i == 0, j == NBLK - 1))
    def _accumulate_tail():
        n_valid = N - (NBLK - 1) * BLK
        a = adjt_ref[...].astype(jnp.bfloat16)
        e = emb_ref[...].astype(jnp.bfloat16)
        acol = jax.lax.broadcasted_iota(jnp.int32, (H, BLK), 1)
        erow = jax.lax.broadcasted_iota(jnp.int32, (BLK, D), 0)
        a = jnp.where(acol < n_valid, a, jnp.bfloat16(0))
        e = jnp.where(erow < n_valid, e, jnp.bfloat16(0))
        stash_ref[j] = a
        lat_ref[...] += jnp.dot(a, e, preferred_element_type=jnp.float32)

    @pl.when(i == 1)
    def _emit():
        out_ref[...] = jax.lax.dot_general(
            stash_ref[j], lat_ref[...].astype(jnp.bfloat16),
            dimension_numbers=(((0,), (0,)), ((), ())),
            preferred_element_type=jnp.float32,
        )


def kernel(adj, embeds):
    adjt = jnp.swapaxes(adj, 0, 1)
    ret = pl.pallas_call(
        _fused_kernel,
        grid=(2, NBLK),
        in_specs=[
            pl.BlockSpec((H, BLK), lambda i, j: (0, (1 - i) * j + i * (NBLK - 1))),
            pl.BlockSpec((BLK, D), lambda i, j: ((1 - i) * j + i * (NBLK - 1), 0)),
        ],
        out_specs=pl.BlockSpec((BLK, D), lambda i, j: (i * j, 0)),
        out_shape=jax.ShapeDtypeStruct((N, D), jnp.float32),
        scratch_shapes=[
            pltpu.VMEM((NBLK, H, BLK), jnp.bfloat16),
            pltpu.VMEM((H, D), jnp.float32),
        ],
    )(adjt, embeds)
    return ret


# speedup vs baseline: 1.0759x; 1.0404x over previous
import jax
import jax.numpy as jnp
from jax.experimental import pallas as pl
from jax.experimental.pallas import tpu as pltpu

N = 100000
H = 64
D = 128
BLK = 12800
HB = BLK // 2
NBLK = (N + BLK - 1) // BLK


def _fused_kernel(a1_ref, a2_ref, e1_ref, e2_ref, out_ref, stash_ref, lat_ref):
    i = pl.program_id(0)
    j = pl.program_id(1)

    @pl.when(jnp.logical_and(i == 0, j == 0))
    def _init():
        lat_ref[...] = jnp.zeros_like(lat_ref)

    @pl.when(jnp.logical_and(i == 0, j < NBLK - 1))
    def _accumulate():
        a1 = a1_ref[...].astype(jnp.bfloat16)
        a2 = a2_ref[...].astype(jnp.bfloat16)
        e1 = e1_ref[...].astype(jnp.bfloat16)
        e2 = e2_ref[...].astype(jnp.bfloat16)
        stash_ref[j, :, :HB] = a1
        stash_ref[j, :, HB:] = a2
        lat_ref[...] += jnp.dot(a1, e1, preferred_element_type=jnp.float32)
        lat_ref[...] += jnp.dot(a2, e2, preferred_element_type=jnp.float32)

    @pl.when(jnp.logical_and(i == 0, j == NBLK - 1))
    def _accumulate_tail():
        n_valid = N - (NBLK - 1) * BLK - HB
        a1 = a1_ref[...].astype(jnp.bfloat16)
        a2 = a2_ref[...].astype(jnp.bfloat16)
        e1 = e1_ref[...].astype(jnp.bfloat16)
        e2 = e2_ref[...].astype(jnp.bfloat16)
        acol = jax.lax.broadcasted_iota(jnp.int32, (H, HB), 1)
        erow = jax.lax.broadcasted_iota(jnp.int32, (HB, D), 0)
        a2 = jnp.where(acol < n_valid, a2, jnp.bfloat16(0))
        e2 = jnp.where(erow < n_valid, e2, jnp.bfloat16(0))
        stash_ref[j, :, :HB] = a1
        stash_ref[j, :, HB:] = a2
        lat_ref[...] += jnp.dot(a1, e1, preferred_element_type=jnp.float32)
        lat_ref[...] += jnp.dot(a2, e2, preferred_element_type=jnp.float32)

    @pl.when(i == 1)
    def _emit():
        out_ref[...] = jax.lax.dot_general(
            stash_ref[j], lat_ref[...].astype(jnp.bfloat16),
            dimension_numbers=(((0,), (0,)), ((), ())),
            preferred_element_type=jnp.float32,
        )


def kernel(adj, embeds):
    adjt = jnp.swapaxes(adj, 0, 1)
    hold = 2 * (NBLK - 1)
    ret = pl.pallas_call(
        _fused_kernel,
        grid=(2, NBLK),
        in_specs=[
            pl.BlockSpec((H, HB), lambda i, j: (0, (1 - i) * 2 * j + i * hold)),
            pl.BlockSpec((H, HB), lambda i, j: (0, (1 - i) * (2 * j + 1) + i * (hold + 1))),
            pl.BlockSpec((HB, D), lambda i, j: ((1 - i) * 2 * j + i * hold, 0)),
            pl.BlockSpec((HB, D), lambda i, j: ((1 - i) * (2 * j + 1) + i * (hold + 1), 0)),
        ],
        out_specs=pl.BlockSpec((BLK, D), lambda i, j: (i * j, 0)),
        out_shape=jax.ShapeDtypeStruct((N, D), jnp.float32),
        scratch_shapes=[
            pltpu.VMEM((NBLK, H, BLK), jnp.bfloat16),
            pltpu.VMEM((H, D), jnp.float32),
        ],
    )(adjt, adjt, embeds, embeds)
    return ret


# speedup vs baseline: 1.0777x; 1.0017x over previous
import jax
import jax.numpy as jnp
from jax.experimental import pallas as pl
from jax.experimental.pallas import tpu as pltpu

N = 100000
H = 64
D = 128
BLK = 12800
NBLK = (N + BLK - 1) // BLK


def _fused_kernel(adjt_ref, emb_ref, out_ref, stash_ref, lat_ref):
    g = pl.program_id(0)

    @pl.when(g == 0)
    def _init():
        lat_ref[...] = jnp.zeros_like(lat_ref)

    @pl.when(g < NBLK - 1)
    def _accumulate():
        a = adjt_ref[...].astype(jnp.bfloat16)
        e = emb_ref[...].astype(jnp.bfloat16)
        stash_ref[g] = a
        lat_ref[...] += jnp.dot(a, e, preferred_element_type=jnp.float32)

    @pl.when(g == NBLK - 1)
    def _accumulate_tail_and_emit():
        n_valid = N - (NBLK - 1) * BLK
        a = adjt_ref[...].astype(jnp.bfloat16)
        e = emb_ref[...].astype(jnp.bfloat16)
        acol = jax.lax.broadcasted_iota(jnp.int32, (H, BLK), 1)
        erow = jax.lax.broadcasted_iota(jnp.int32, (BLK, D), 0)
        a = jnp.where(acol < n_valid, a, jnp.bfloat16(0))
        e = jnp.where(erow < n_valid, e, jnp.bfloat16(0))
        lat = lat_ref[...] + jnp.dot(a, e, preferred_element_type=jnp.float32)
        lat_ref[...] = lat
        out_ref[...] = jax.lax.dot_general(
            a, lat.astype(jnp.bfloat16),
            dimension_numbers=(((0,), (0,)), ((), ())),
            preferred_element_type=jnp.float32,
        )

    @pl.when(g >= NBLK)
    def _emit():
        out_ref[...] = jax.lax.dot_general(
            stash_ref[g - NBLK], lat_ref[...].astype(jnp.bfloat16),
            dimension_numbers=(((0,), (0,)), ((), ())),
            preferred_element_type=jnp.float32,
        )


def kernel(adj, embeds):
    adjt = jnp.swapaxes(adj, 0, 1)
    ret = pl.pallas_call(
        _fused_kernel,
        grid=(2 * NBLK - 1,),
        in_specs=[
            pl.BlockSpec((H, BLK), lambda g: (0, jnp.minimum(g, NBLK - 1))),
            pl.BlockSpec((BLK, D), lambda g: (jnp.minimum(g, NBLK - 1), 0)),
        ],
        out_specs=pl.BlockSpec(
            (BLK, D),
            lambda g: (jnp.where(g < NBLK, NBLK - 1, g - NBLK), 0),
        ),
        out_shape=jax.ShapeDtypeStruct((N, D), jnp.float32),
        scratch_shapes=[
            pltpu.VMEM((NBLK - 1, H, BLK), jnp.bfloat16),
            pltpu.VMEM((H, D), jnp.float32),
        ],
    )(adjt, embeds)
    return ret
